# baseline (speedup 1.0000x reference)
import functools

import jax
import jax.numpy as jnp
from jax import lax
from jax.experimental import pallas as pl
from jax.experimental.pallas import tpu as pltpu
from jax.experimental.pallas import tpu_sc as plsc

_MARGIN_ML = 0.2
_MARGIN_CL = 1.0
_W_ML = 2.0
_W_CL = 2.0

_B, _H, _P = 8192, 16, 256
_K = 1024
_M = 4096
_D = _H * _P

_NC, _NS, _L = 2, 16, 16
_NW = _NC * _NS

_CHUNK = 4
_RPB = 128


def _tc_cluster_body(aidx_ref, z_ref, c_ref, out_ref):
    i = pl.program_id(0)

    @pl.when(i == 0)
    def _init():
        out_ref[...] = jnp.zeros_like(out_ref)

    acc = jnp.zeros((1, _H), jnp.float32)
    for k in range(_RPB):
        ck = c_ref[pl.ds(aidx_ref[i * _RPB + k], 1)]
        d = z_ref[k] - ck[0]
        acc = acc + jnp.sum(d * d, axis=-1)[None, :]
    out_ref[...] += acc


@jax.jit
def _tc_cluster(aidx, z, centroids):
    grid_spec = pltpu.PrefetchScalarGridSpec(
        num_scalar_prefetch=1,
        grid=(_B // _RPB,),
        in_specs=[
            pl.BlockSpec((_RPB, _H, _P), lambda i, aidx: (i, 0, 0)),
            pl.BlockSpec((_K, _H, _P), lambda i, aidx: (0, 0, 0)),
        ],
        out_specs=pl.BlockSpec((1, _H), lambda i, aidx: (0, 0)),
    )
    head_sums = pl.pallas_call(
        _tc_cluster_body,
        grid_spec=grid_spec,
        out_shape=jax.ShapeDtypeStruct((1, _H), jnp.float32),
        compiler_params=pltpu.CompilerParams(
            dimension_semantics=("arbitrary",)),
    )(aidx, z, centroids)
    return head_sums


_GRP = 4
_ROWS = 2 * _GRP


def _rowpair_dist(buf, c):
    lane = jnp.arange(_L, dtype=jnp.int32)
    d16 = jnp.zeros((_L,), jnp.float32)
    for h in range(_H):
        s = jnp.zeros((_L,), jnp.float32)
        for v in range(_P // _L):
            sl = pl.ds(v * _L, _L)
            d = buf[c, h, sl] - buf[c + _GRP, h, sl]
            s = s + d * d
        d16 = jnp.where(lane == h, jnp.sum(s), d16)
    return d16


def _sc_body(z_hbm, ml_hbm, cl_hbm,
             out_hbm, idx_v, buf0, buf1, out_v, sem0, sem1):
    wid = lax.axis_index("s") * _NC + lax.axis_index("c")
    idx_per_w = 2 * _M // _NW
    n_chunks = idx_per_w // _ROWS

    def pair_phase(ilv_hbm, reduce_fn):
        pltpu.sync_copy(ilv_hbm.at[pl.ds(wid * idx_per_w, idx_per_w)], idx_v)

        def issue(ch, buf, sem):
            pltpu.async_copy(
                z_hbm.at[idx_v.at[pl.ds(ch * _ROWS, _ROWS)]], buf, sem)

        def wait(buf, sem):
            pltpu.make_async_copy(
                z_hbm.at[idx_v.at[pl.ds(0, _ROWS)]], buf, sem).wait()

        def compute(buf, acc):
            def row_body(c, a):
                return reduce_fn(a, _rowpair_dist(buf, c))
            return lax.fori_loop(0, _GRP, row_body, acc)

        issue(0, buf0, sem0)

        def body(ch2, acc):
            base = 2 * ch2
            issue(base + 1, buf1, sem1)
            wait(buf0, sem0)
            acc = compute(buf0, acc)
            issue(jnp.minimum(base + 2, n_chunks - 1), buf0, sem0)
            wait(buf1, sem1)
            return compute(buf1, acc)

        acc = lax.fori_loop(0, n_chunks // 2, body,
                            jnp.zeros((_L,), jnp.float32))
        wait(buf0, sem0)
        return acc

    out_v[0] = pair_phase(
        ml_hbm, lambda acc, d: acc + jnp.maximum(d - _MARGIN_ML, 0.0))
    out_v[1] = pair_phase(
        cl_hbm, lambda acc, d: acc + jnp.maximum(_MARGIN_CL - d, 0.0))

    pltpu.sync_copy(out_v, out_hbm.at[wid])


@jax.jit
def _sc_pairs(zf, ml_ilv, cl_ilv):
    mesh = plsc.VectorSubcoreMesh(core_axis_name="c", subcore_axis_name="s")
    buf_t = pltpu.VMEM((_ROWS, _H, _P), jnp.float32)
    return pl.kernel(
        _sc_body,
        out_type=jax.ShapeDtypeStruct((_NW, 2, _L), jnp.float32),
        mesh=mesh,
        scratch_types=[
            pltpu.VMEM((2 * _M // _NW,), jnp.int32),
            buf_t, buf_t,
            pltpu.VMEM((2, _L), jnp.float32),
            pltpu.SemaphoreType.DMA,
            pltpu.SemaphoreType.DMA,
        ],
        compiler_params=pltpu.CompilerParams(needs_layout_passes=False),
    )(zf, ml_ilv, cl_ilv)


def _interleave(links):
    idx = links.astype(jnp.int32)
    return jnp.concatenate(
        [idx[:, 0].reshape(-1, 1, _GRP), idx[:, 1].reshape(-1, 1, _GRP)],
        axis=1).reshape(-1)


def kernel(z, centroids, assignments, must_links, cannot_links):
    aidx = assignments.astype(jnp.int32)
    ml_ilv = _interleave(must_links)
    cl_ilv = _interleave(cannot_links)

    cluster_heads = _tc_cluster(aidx, z, centroids)
    pair_parts = _sc_pairs(z, ml_ilv, cl_ilv)

    sums = pair_parts.sum(axis=0)
    loss_cluster = jnp.sum(cluster_heads) / (_B * _H)
    loss_ml = jnp.min(sums[0] / _M) * _W_ML
    loss_cl = jnp.min(sums[1] / _M) * _W_CL
    return loss_cluster, loss_ml + loss_cl


# speedup vs baseline: 1.2470x; 1.2470x over previous
import functools

import jax
import jax.numpy as jnp
from jax import lax
from jax.experimental import pallas as pl
from jax.experimental.pallas import tpu as pltpu
from jax.experimental.pallas import tpu_sc as plsc

_MARGIN_ML = 0.2
_MARGIN_CL = 1.0
_W_ML = 2.0
_W_CL = 2.0

_B, _H, _P = 8192, 16, 256
_K = 1024
_M = 4096
_D = _H * _P

_NC, _NS, _L = 2, 16, 16
_NW = _NC * _NS

_CHUNK = 4
_RPB = 128


_CLTC = 2048
_PPS = _CLTC // (_B // _RPB)


def _tc_cluster_body(aidx_ref, ci_ref, cj_ref, z_ref, c_ref, *rest):
    zi_refs = rest[:_PPS]
    zj_refs = rest[_PPS:2 * _PPS]
    out_ref, cl_ref = rest[2 * _PPS], rest[2 * _PPS + 1]
    i = pl.program_id(0)

    @pl.when(i == 0)
    def _init():
        out_ref[...] = jnp.zeros_like(out_ref)
        cl_ref[...] = jnp.zeros_like(cl_ref)

    acc = jnp.zeros((1, _H), jnp.float32)
    for k in range(_RPB):
        ck = c_ref[pl.ds(aidx_ref[i * _RPB + k], 1)]
        d = z_ref[k] - ck[0]
        acc = acc + jnp.sum(d * d, axis=-1)[None, :]
    out_ref[...] += acc

    acc_cl = jnp.zeros((1, _H), jnp.float32)
    for p in range(_PPS):
        d = zi_refs[p][0] - zj_refs[p][0]
        dist = jnp.sum(d * d, axis=-1)[None, :]
        acc_cl = acc_cl + jnp.maximum(_MARGIN_CL - dist, 0.0)
    cl_ref[...] += acc_cl


def _imap(k, i, aidx, ci, cj):
    return (ci[i * _PPS + k], 0, 0)


def _jmap(k, i, aidx, ci, cj):
    return (cj[i * _PPS + k], 0, 0)


@jax.jit
def _tc_cluster(aidx, ci, cj, z, centroids):
    row_spec = pl.BlockSpec((1, _H, _P))
    grid_spec = pltpu.PrefetchScalarGridSpec(
        num_scalar_prefetch=3,
        grid=(_B // _RPB,),
        in_specs=[
            pl.BlockSpec((_RPB, _H, _P), lambda i, *_: (i, 0, 0)),
            pl.BlockSpec((_K, _H, _P), lambda i, *_: (0, 0, 0)),
        ]
        + [pl.BlockSpec((1, _H, _P), functools.partial(_imap, k))
           for k in range(_PPS)]
        + [pl.BlockSpec((1, _H, _P), functools.partial(_jmap, k))
           for k in range(_PPS)],
        out_specs=[pl.BlockSpec((1, _H), lambda i, *_: (0, 0)),
                   pl.BlockSpec((1, _H), lambda i, *_: (0, 0))],
    )
    head_sums, cl_sums = pl.pallas_call(
        _tc_cluster_body,
        grid_spec=grid_spec,
        out_shape=[jax.ShapeDtypeStruct((1, _H), jnp.float32),
                   jax.ShapeDtypeStruct((1, _H), jnp.float32)],
        compiler_params=pltpu.CompilerParams(
            dimension_semantics=("arbitrary",)),
    )(aidx, ci, cj, z, centroids, *([z] * (2 * _PPS)))
    return head_sums, cl_sums


_GRP = 4
_ROWS = 2 * _GRP


def _rowpair_dist(buf, c):
    lane = jnp.arange(_L, dtype=jnp.int32)
    d16 = jnp.zeros((_L,), jnp.float32)
    for h in range(_H):
        s = jnp.zeros((_L,), jnp.float32)
        for v in range(_P // _L):
            sl = pl.ds(v * _L, _L)
            d = buf[c, h, sl] - buf[c + _GRP, h, sl]
            s = s + d * d
        d16 = jnp.where(lane == h, jnp.sum(s), d16)
    return d16


def _sc_body(z_hbm, ml_hbm, cl_hbm,
             out_hbm, idx_v, buf0, buf1, out_v, sem0, sem1):
    wid = lax.axis_index("s") * _NC + lax.axis_index("c")

    def pair_phase(ilv_hbm, idx_per_w, reduce_fn):
        n_chunks = idx_per_w // _ROWS
        pltpu.sync_copy(ilv_hbm.at[pl.ds(wid * idx_per_w, idx_per_w)],
                        idx_v.at[pl.ds(0, idx_per_w)])

        def issue(ch, buf, sem):
            pltpu.async_copy(
                z_hbm.at[idx_v.at[pl.ds(ch * _ROWS, _ROWS)]], buf, sem)

        def wait(buf, sem):
            pltpu.make_async_copy(
                z_hbm.at[idx_v.at[pl.ds(0, _ROWS)]], buf, sem).wait()

        def compute(buf, acc):
            def row_body(c, a):
                return reduce_fn(a, _rowpair_dist(buf, c))
            return lax.fori_loop(0, _GRP, row_body, acc)

        issue(0, buf0, sem0)

        def body(ch2, acc):
            base = 2 * ch2
            issue(base + 1, buf1, sem1)
            wait(buf0, sem0)
            acc = compute(buf0, acc)
            issue(jnp.minimum(base + 2, n_chunks - 1), buf0, sem0)
            wait(buf1, sem1)
            return compute(buf1, acc)

        acc = lax.fori_loop(0, n_chunks // 2, body,
                            jnp.zeros((_L,), jnp.float32))
        wait(buf0, sem0)
        return acc

    out_v[0] = pair_phase(
        ml_hbm, 2 * _M // _NW,
        lambda acc, d: acc + jnp.maximum(d - _MARGIN_ML, 0.0))
    out_v[1] = pair_phase(
        cl_hbm, 2 * (_M - _CLTC) // _NW,
        lambda acc, d: acc + jnp.maximum(_MARGIN_CL - d, 0.0))

    pltpu.sync_copy(out_v, out_hbm.at[wid])


@jax.jit
def _sc_pairs(zf, ml_ilv, cl_ilv):
    mesh = plsc.VectorSubcoreMesh(core_axis_name="c", subcore_axis_name="s")
    buf_t = pltpu.VMEM((_ROWS, _H, _P), jnp.float32)
    return pl.kernel(
        _sc_body,
        out_type=jax.ShapeDtypeStruct((_NW, 2, _L), jnp.float32),
        mesh=mesh,
        scratch_types=[
            pltpu.VMEM((2 * _M // _NW,), jnp.int32),
            buf_t, buf_t,
            pltpu.VMEM((2, _L), jnp.float32),
            pltpu.SemaphoreType.DMA,
            pltpu.SemaphoreType.DMA,
        ],
        compiler_params=pltpu.CompilerParams(needs_layout_passes=False),
    )(zf, ml_ilv, cl_ilv)


def _interleave(links):
    idx = links.astype(jnp.int32)
    return jnp.concatenate(
        [idx[:, 0].reshape(-1, 1, _GRP), idx[:, 1].reshape(-1, 1, _GRP)],
        axis=1).reshape(-1)


def kernel(z, centroids, assignments, must_links, cannot_links):
    aidx = assignments.astype(jnp.int32)
    ml_ilv = _interleave(must_links)
    cl_ilv = _interleave(cannot_links[:_M - _CLTC])
    ci = cannot_links[_M - _CLTC:, 0].astype(jnp.int32)
    cj = cannot_links[_M - _CLTC:, 1].astype(jnp.int32)

    cluster_heads, cl_tc = _tc_cluster(aidx, ci, cj, z, centroids)
    pair_parts = _sc_pairs(z, ml_ilv, cl_ilv)

    sums = pair_parts.sum(axis=0)
    loss_cluster = jnp.sum(cluster_heads) / (_B * _H)
    loss_ml = jnp.min(sums[0] / _M) * _W_ML
    loss_cl = jnp.min((sums[1] + cl_tc[0]) / _M) * _W_CL
    return loss_cluster, loss_ml + loss_cl
